# baseline (speedup 1.0000x reference)
import jax
import jax.numpy as jnp
from jax import lax
from jax.experimental import pallas as pl
from jax.experimental.pallas import tpu as pltpu
from jax.experimental.pallas import tpu_sc as plsc

_GDN = lax.GatherDimensionNumbers(
    offset_dims=(), collapsed_slice_dims=(0,), start_index_map=(0,))


def _vpermute(v, idx):
    return lax.gather(v, idx[:, None], dimension_numbers=_GDN,
                      slice_sizes=(1,),
                      mode=lax.GatherScatterMode.PROMISE_IN_BOUNDS)


D = 128
H = 2
R = 3
EPS = 1e-10
SB = 32
NSUB = 16
NPAD = 10240


def _stage_a_body(x_ref, w_ref, q_ref, k_ref, v_ref):
    big = jnp.dot(x_ref[...], w_ref[...].T, preferred_element_type=jnp.float32)
    q_ref[0] = big[:, 0:128]
    q_ref[1] = big[:, 128:256]
    k_ref[0] = big[:, 256:640]
    k_ref[1] = big[:, 640:1024]
    v_ref[0] = big[:, 1024:1408]
    v_ref[1] = big[:, 1408:1792]


def _stage_a(x, wbig, n, bn):
    return pl.pallas_call(
        _stage_a_body,
        grid=(n // bn,),
        in_specs=[
            pl.BlockSpec((bn, D), lambda i: (i, 0)),
            pl.BlockSpec((H * (1 + 2 * R) * D, D), lambda i: (0, 0)),
        ],
        out_specs=[
            pl.BlockSpec((H, bn, D), lambda i: (0, i, 0)),
            pl.BlockSpec((H, bn, R * D), lambda i: (0, i, 0)),
            pl.BlockSpec((H, bn, R * D), lambda i: (0, i, 0)),
        ],
        out_shape=[
            jax.ShapeDtypeStruct((H, n, D), jnp.float32),
            jax.ShapeDtypeStruct((H, n, R * D), jnp.float32),
            jax.ShapeDtypeStruct((H, n, R * D), jnp.float32),
        ],
    )(x, wbig)


def _make_sc_kernel(n, e2):
    eb = e2 // NSUB
    nblocks = eb // SB
    assert nblocks % 2 == 0
    nhalf = nblocks // 2
    npart = NPAD // NSUB
    drain_rows = 32
    ndrain = npart // drain_rows

    mesh = plsc.VectorSubcoreMesh(core_axis_name="c", subcore_axis_name="s")

    def body(q_hbm, k_hbm, v_hbm, pack_hbm, zero_hbm, zflat_hbm, out_hbm,
             idx0, idx1, kvidx0, kvidx1, qidx0, qidx1, sidx0, sidx1,
             q0, q1, k0, k1, v0, v1, nb0, nb1, dbuf, tmp, acc, den_sh,
             gk0, gq0, gv0, sv0, sd0, gk1, gq1, gv1, sv1, sd1):
        c = lax.axis_index("c")
        s = lax.axis_index("s")
        lane = lax.iota(jnp.int32, 16)
        slot0 = (idx0, kvidx0, qidx0, sidx0, q0, k0, v0, nb0,
                 gk0, gq0, gv0, sv0, sd0)
        slot1 = (idx1, kvidx1, qidx1, sidx1, q1, k1, v1, nb1,
                 gk1, gq1, gv1, sv1, sd1)
        pltpu.sync_copy(zero_hbm.at[pl.ds(s * npart, npart)],
                        acc.at[pl.ds(s * npart, npart)])

        @pl.when(s == 0)
        def _():
            pltpu.sync_copy(zflat_hbm, den_sh)

        plsc.subcore_barrier()
        bbase = s * nblocks
        qbase = c * n
        kvbase = c * (R * n)

        def prefetch(bt, sl, wait_sc):
            (idxb, kvidx, qidx, sidx, q_r, k_r, v_r, nbuf,
             gk, gq, gv, sv, sd) = sl

            @pl.when(wait_sc)
            def _():
                pltpu.make_async_copy(v_r, acc.at[sidx], sv).wait()
                pltpu.make_async_copy(nbuf, den_sh.at[sidx], sd).wait()

            pltpu.sync_copy(pack_hbm.at[bbase + bt], idxb)
            for j in range(SB // 16):
                jj = pl.ds(j * 16, 16)
                kvidx[jj] = idxb[0, jj] + kvbase
                qidx[jj] = idxb[1, jj] + qbase
                sidx[jj] = idxb[2, jj]
            pltpu.async_copy(k_hbm.at[kvidx], k_r, gk)
            pltpu.async_copy(q_hbm.at[qidx], q_r, gq)
            pltpu.async_copy(v_hbm.at[kvidx], v_r, gv)

        def process(sl):
            (idxb, kvidx, qidx, sidx, q_rows, k_rows, v_rows, nbuf,
             gk, gq, gv, sv, sd) = sl
            pltpu.make_async_copy(k_hbm.at[kvidx], k_rows, gk).wait()
            pltpu.make_async_copy(q_hbm.at[qidx], q_rows, gq).wait()
            pltpu.make_async_copy(v_hbm.at[kvidx], v_rows, gv).wait()

            for g in range(SB // 16):
                base = g * 16
                numers = jnp.zeros((16,), jnp.float32)
                for j in range(16):
                    ei = base + j
                    acc_v = q_rows[ei, pl.ds(0, 16)] * k_rows[ei, pl.ds(0, 16)]
                    for k in range(1, 8):
                        acc_v = acc_v + (q_rows[ei, pl.ds(16 * k, 16)]
                                         * k_rows[ei, pl.ds(16 * k, 16)])
                    for k in (1, 2, 4, 8):
                        acc_v = acc_v + _vpermute(acc_v, lane ^ k)
                    rv = jnp.maximum(acc_v, 0.0)
                    nv = rv * rv * (1.0 / 256.0) + EPS
                    numers = numers + jnp.where(lane == j, nv, 0.0)
                    for k in range(8):
                        kk = pl.ds(16 * k, 16)
                        v_rows[ei, kk] = nv * v_rows[ei, kk]
                nbuf[pl.ds(base, 16)] = numers

            pltpu.async_copy(v_rows, acc.at[sidx], sv, add=True)
            pltpu.async_copy(nbuf, den_sh.at[sidx], sd, add=True)

        prefetch(0, slot0, False)

        def iter2(i, carry):
            b0 = 2 * i
            prefetch(b0 + 1, slot1, i > 0)
            process(slot0)

            @pl.when(i < nhalf - 1)
            def _():
                prefetch(b0 + 2, slot0, True)

            process(slot1)
            return carry

        lax.fori_loop(0, nhalf, iter2, 0)
        pltpu.make_async_copy(v0, acc.at[sidx0], sv0).wait()
        pltpu.make_async_copy(nb0, den_sh.at[sidx0], sd0).wait()
        pltpu.make_async_copy(v1, acc.at[sidx1], sv1).wait()
        pltpu.make_async_copy(nb1, den_sh.at[sidx1], sd1).wait()
        plsc.subcore_barrier()

        def drain(ch, carry):
            n0 = s * npart + ch * drain_rows
            pltpu.sync_copy(acc.at[pl.ds(n0, drain_rows)], tmp)
            pltpu.sync_copy(den_sh.at[pl.ds(n0, drain_rows)], dbuf)
            for half in range(drain_rows // 16):
                dv = dbuf[pl.ds(half * 16, 16)]
                inv = 1.0 / jnp.where(dv > 0.0, dv, 1.0)

                def row(r, carry2):
                    bc = _vpermute(inv, jnp.full((16,), r, jnp.int32))
                    rr = half * 16 + r
                    for k in range(8):
                        kk = pl.ds(16 * k, 16)
                        tmp[rr, kk] = tmp[rr, kk] * bc
                    return carry2

                lax.fori_loop(0, 16, row, 0)
            pltpu.sync_copy(tmp, out_hbm.at[c, pl.ds(n0, drain_rows)])
            return carry

        lax.fori_loop(0, ndrain, drain, 0)

    return pl.kernel(
        body,
        out_type=jax.ShapeDtypeStruct((H, NPAD, D), jnp.float32),
        mesh=mesh,
        compiler_params=pltpu.CompilerParams(needs_layout_passes=False),
        scratch_types=[
            pltpu.VMEM((3, SB), jnp.int32),
            pltpu.VMEM((3, SB), jnp.int32),
            pltpu.VMEM((SB,), jnp.int32),
            pltpu.VMEM((SB,), jnp.int32),
            pltpu.VMEM((SB,), jnp.int32),
            pltpu.VMEM((SB,), jnp.int32),
            pltpu.VMEM((SB,), jnp.int32),
            pltpu.VMEM((SB,), jnp.int32),
            pltpu.VMEM((SB, D), jnp.float32),
            pltpu.VMEM((SB, D), jnp.float32),
            pltpu.VMEM((SB, D), jnp.float32),
            pltpu.VMEM((SB, D), jnp.float32),
            pltpu.VMEM((SB, D), jnp.float32),
            pltpu.VMEM((SB, D), jnp.float32),
            pltpu.VMEM((SB,), jnp.float32),
            pltpu.VMEM((SB,), jnp.float32),
            pltpu.VMEM((32,), jnp.float32),
            pltpu.VMEM((32, D), jnp.float32),
            pltpu.VMEM_SHARED((NPAD, D), jnp.float32),
            pltpu.VMEM_SHARED((NPAD,), jnp.float32),
            pltpu.SemaphoreType.DMA,
            pltpu.SemaphoreType.DMA,
            pltpu.SemaphoreType.DMA,
            pltpu.SemaphoreType.DMA,
            pltpu.SemaphoreType.DMA,
            pltpu.SemaphoreType.DMA,
            pltpu.SemaphoreType.DMA,
            pltpu.SemaphoreType.DMA,
            pltpu.SemaphoreType.DMA,
            pltpu.SemaphoreType.DMA,
        ],
    )


def _stage_c_body(z_ref, wo_ref, o_ref):
    wo = wo_ref[...]
    o_ref[...] = (
        jnp.dot(z_ref[0], wo[:, 0:128].T, preferred_element_type=jnp.float32)
        + jnp.dot(z_ref[1], wo[:, 128:256].T, preferred_element_type=jnp.float32))


def _stage_c(z, wo, n, bn):
    return pl.pallas_call(
        _stage_c_body,
        grid=((n + bn - 1) // bn,),
        in_specs=[
            pl.BlockSpec((H, bn, D), lambda i: (0, i, 0)),
            pl.BlockSpec((D, H * D), lambda i: (0, 0)),
        ],
        out_specs=pl.BlockSpec((bn, D), lambda i: (i, 0)),
        out_shape=jax.ShapeDtypeStruct((n, D), jnp.float32),
    )(z, wo)


@jax.jit
def kernel(node_feature, edge_index, edge_type, WQ, WK, WV, WO):
    n, d = node_feature.shape
    e = edge_index.shape[1]
    assert d == D

    parts = [WQ[0:D], WQ[D:2 * D]]
    for c in range(H):
        for r in range(R):
            parts.append(WK[r, c * D:(c + 1) * D])
    for c in range(H):
        for r in range(R):
            parts.append(WV[r, c * D:(c + 1) * D])
    wbig = jnp.concatenate(parts, axis=0)

    q_out, k_out, v_out = _stage_a(node_feature, wbig, n, 400)
    q_tab = q_out.reshape(H * n, D)
    k_tab = k_out.reshape(H * R * n, D)
    v_tab = v_out.reshape(H * R * n, D)

    src = edge_index[0].astype(jnp.int32)
    dst = edge_index[1].astype(jnp.int32)
    ty = edge_type.astype(jnp.int32)
    grain = NSUB * 2 * SB
    e2 = ((e + grain - 1) // grain) * grain
    pad = e2 - e
    kv_row = jnp.concatenate([src * R + ty, jnp.zeros((pad,), jnp.int32)])
    q_row = jnp.concatenate([dst, jnp.zeros((pad,), jnp.int32)])
    s_row = jnp.concatenate([dst, jnp.full((pad,), NPAD - 1, jnp.int32)])
    pack = (jnp.stack([kv_row, q_row, s_row], axis=0)
            .reshape(3, e2 // SB, SB).transpose(1, 0, 2))
    zeros = jnp.zeros((NPAD, D), jnp.float32)
    zflat = jnp.zeros((NPAD,), jnp.float32)

    sc = _make_sc_kernel(n, e2)
    z = sc(q_tab, k_tab, v_tab, pack, zeros, zflat)

    return _stage_c(z, WO, n, 512)


# speedup vs baseline: 1.4560x; 1.4560x over previous
import jax
import jax.numpy as jnp
from jax import lax
from jax.experimental import pallas as pl
from jax.experimental.pallas import tpu as pltpu
from jax.experimental.pallas import tpu_sc as plsc

_GDN = lax.GatherDimensionNumbers(
    offset_dims=(), collapsed_slice_dims=(0,), start_index_map=(0,))


def _vpermute(v, idx):
    return lax.gather(v, idx[:, None], dimension_numbers=_GDN,
                      slice_sizes=(1,),
                      mode=lax.GatherScatterMode.PROMISE_IN_BOUNDS)


D = 128
H = 2
R = 3
EPS = 1e-10


def _stage_a_body(x_ref, w_ref, q_ref, k_ref, v_ref):
    big = jnp.dot(x_ref[...], w_ref[...].T, preferred_element_type=jnp.float32)
    q_ref[0] = big[:, 0:128]
    q_ref[1] = big[:, 128:256]
    k_ref[0] = big[:, 256:640]
    k_ref[1] = big[:, 640:1024]
    v_ref[0] = big[:, 1024:1408]
    v_ref[1] = big[:, 1408:1792]


def _stage_a(x, wbig, n, bn):
    return pl.pallas_call(
        _stage_a_body,
        grid=(n // bn,),
        in_specs=[
            pl.BlockSpec((bn, D), lambda i: (i, 0)),
            pl.BlockSpec((H * (1 + 2 * R) * D, D), lambda i: (0, 0)),
        ],
        out_specs=[
            pl.BlockSpec((H, bn, D), lambda i: (0, i, 0)),
            pl.BlockSpec((H, bn, R * D), lambda i: (0, i, 0)),
            pl.BlockSpec((H, bn, R * D), lambda i: (0, i, 0)),
        ],
        out_shape=[
            jax.ShapeDtypeStruct((H, n, D), jnp.float32),
            jax.ShapeDtypeStruct((H, n, R * D), jnp.float32),
            jax.ShapeDtypeStruct((H, n, R * D), jnp.float32),
        ],
    )(x, wbig)


def _make_sc_kernel(n, npad, e):
    nsub = 16
    eb = e // nsub
    sb = 80
    assert eb % sb == 0
    nblocks = eb // sb
    npart = npad // nsub
    assert npart % 8 == 0
    dnr = npad // 128
    drain_rows = 32
    ndrain = npart // drain_rows

    mesh = plsc.VectorSubcoreMesh(core_axis_name="c", subcore_axis_name="s")

    def body(q_hbm, k_hbm, v_hbm, pack_hbm, zero_hbm, out_hbm,
             idxb, qidx, kvidx, sidx, idbuf, q_rows, k_rows,
             v_rows, den_l, tmp, acc, den_sh, sem1, sem2, sem3):
        c = lax.axis_index("c")
        s = lax.axis_index("s")
        lane = lax.iota(jnp.int32, 16)
        pltpu.sync_copy(zero_hbm.at[pl.ds(s * npart, npart)],
                        acc.at[pl.ds(s * npart, npart)])
        pltpu.sync_copy(zero_hbm.at[pl.ds(0, dnr)], den_l)

        @pl.when(s == 0)
        def _():
            pltpu.sync_copy(zero_hbm.at[pl.ds(0, dnr)], den_sh)

        for g in range(dnr // 16):
            idbuf[pl.ds(16 * g, 16)] = lane + 16 * g
        plsc.subcore_barrier()
        bbase = s * nblocks
        qbase = c * n
        kvbase = c * (R * n)

        def block(b, carry):
            pltpu.sync_copy(pack_hbm.at[bbase + b], idxb)
            for j in range(sb // 16):
                jj = pl.ds(j * 16, 16)
                dsts = idxb[1, jj]
                kvidx[jj] = idxb[0, jj] + kvbase
                qidx[jj] = dsts + qbase
                sidx[jj] = dsts
            g1 = pltpu.async_copy(k_hbm.at[kvidx], k_rows, sem1)
            g2 = pltpu.async_copy(q_hbm.at[qidx], q_rows, sem2)
            g3 = pltpu.async_copy(v_hbm.at[kvidx], v_rows, sem3)
            g1.wait()
            g2.wait()
            g3.wait()

            def group(g, carry2):
                base = g * 16
                numers = jnp.zeros((16,), jnp.float32)
                for j in range(16):
                    ei = base + j
                    acc_v = q_rows[ei, pl.ds(0, 16)] * k_rows[ei, pl.ds(0, 16)]
                    for k in range(1, 8):
                        acc_v = acc_v + (q_rows[ei, pl.ds(16 * k, 16)]
                                         * k_rows[ei, pl.ds(16 * k, 16)])
                    for k in (1, 2, 4, 8):
                        acc_v = acc_v + _vpermute(acc_v, lane ^ k)
                    rv = jnp.maximum(acc_v, 0.0)
                    nv = rv * rv * (1.0 / 256.0) + EPS
                    numers = numers + jnp.where(lane == j, nv, 0.0)
                    for k in range(8):
                        kk = pl.ds(16 * k, 16)
                        v_rows[ei, kk] = nv * v_rows[ei, kk]
                dsts = sidx[pl.ds(base, 16)]
                plsc.addupdate_scatter(
                    den_l, [lax.shift_right_logical(dsts, 7),
                            jnp.bitwise_and(dsts, 127)], numers)
                return carry2

            lax.fori_loop(0, sb // 16, group, 0)
            pltpu.sync_copy(v_rows, acc.at[sidx], add=True)
            return carry

        lax.fori_loop(0, nblocks, block, 0)
        plsc.subcore_barrier()
        pltpu.sync_copy(den_l, den_sh.at[idbuf], add=True)
        plsc.subcore_barrier()
        pltpu.sync_copy(den_sh, den_l)

        def drain(ch, carry):
            n0 = s * npart + ch * drain_rows
            pltpu.sync_copy(acc.at[pl.ds(n0, drain_rows)], tmp)

            def row(r, carry2):
                node = n0 + r
                dv = plsc.load_gather(
                    den_l, [jnp.full((16,), lax.shift_right_logical(node, 7),
                                     jnp.int32),
                            jnp.full((16,), jnp.bitwise_and(node, 127),
                                     jnp.int32)])
                inv = 1.0 / jnp.where(dv > 0.0, dv, 1.0)
                for k in range(8):
                    kk = pl.ds(16 * k, 16)
                    tmp[r, kk] = tmp[r, kk] * inv
                return carry2

            lax.fori_loop(0, drain_rows, row, 0)
            pltpu.sync_copy(tmp, out_hbm.at[c, pl.ds(n0, drain_rows)])
            return carry

        lax.fori_loop(0, ndrain, drain, 0)

    return pl.kernel(
        body,
        out_type=jax.ShapeDtypeStruct((H, npad, D), jnp.float32),
        mesh=mesh,
        compiler_params=pltpu.CompilerParams(needs_layout_passes=False),
        scratch_types=[
            pltpu.VMEM((2, sb), jnp.int32),
            pltpu.VMEM((sb,), jnp.int32),
            pltpu.VMEM((sb,), jnp.int32),
            pltpu.VMEM((sb,), jnp.int32),
            pltpu.VMEM((npad // 128,), jnp.int32),
            pltpu.VMEM((sb, D), jnp.float32),
            pltpu.VMEM((sb, D), jnp.float32),
            pltpu.VMEM((sb, D), jnp.float32),
            pltpu.VMEM((npad // 128, 128), jnp.float32),
            pltpu.VMEM((32, D), jnp.float32),
            pltpu.VMEM_SHARED((npad, D), jnp.float32),
            pltpu.VMEM_SHARED((npad // 128, 128), jnp.float32),
            pltpu.SemaphoreType.DMA,
            pltpu.SemaphoreType.DMA,
            pltpu.SemaphoreType.DMA,
        ],
    )


def _stage_c_body(z_ref, wo_ref, o_ref):
    wo = wo_ref[...]
    o_ref[...] = (
        jnp.dot(z_ref[0], wo[:, 0:128].T, preferred_element_type=jnp.float32)
        + jnp.dot(z_ref[1], wo[:, 128:256].T, preferred_element_type=jnp.float32))


def _stage_c(z, wo, n, bn):
    return pl.pallas_call(
        _stage_c_body,
        grid=((n + bn - 1) // bn,),
        in_specs=[
            pl.BlockSpec((H, bn, D), lambda i: (0, i, 0)),
            pl.BlockSpec((D, H * D), lambda i: (0, 0)),
        ],
        out_specs=pl.BlockSpec((bn, D), lambda i: (i, 0)),
        out_shape=jax.ShapeDtypeStruct((n, D), jnp.float32),
    )(z, wo)


@jax.jit
def kernel(node_feature, edge_index, edge_type, WQ, WK, WV, WO):
    n, d = node_feature.shape
    e = edge_index.shape[1]
    assert d == D

    parts = [WQ[0:D], WQ[D:2 * D]]
    for c in range(H):
        for r in range(R):
            parts.append(WK[r, c * D:(c + 1) * D])
    for c in range(H):
        for r in range(R):
            parts.append(WV[r, c * D:(c + 1) * D])
    wbig = jnp.concatenate(parts, axis=0)

    q_out, k_out, v_out = _stage_a(node_feature, wbig, n, 400)
    q_tab = q_out.reshape(H * n, D)
    k_tab = k_out.reshape(H * R * n, D)
    v_tab = v_out.reshape(H * R * n, D)

    src = edge_index[0].astype(jnp.int32)
    dst = edge_index[1].astype(jnp.int32)
    ty = edge_type.astype(jnp.int32)
    sb = 80
    pack = jnp.stack([(src * R + ty).reshape(e // sb, sb),
                      dst.reshape(e // sb, sb)], axis=1)
    npad = 10240
    zeros = jnp.zeros((npad, D), jnp.float32)

    sc = _make_sc_kernel(n, npad, e)
    z = sc(q_tab, k_tab, v_tab, pack, zeros)

    return _stage_c(z, WO, n, 512)
